# Initial kernel scaffold; baseline (speedup 1.0000x reference)
import jax
import jax.numpy as jnp
from jax.experimental import pallas as pl


def kernel(ray_pts, k0):
    raise NotImplementedError("write your pallas kernel here")


# speedup vs baseline: 1.1009x; 1.1009x over previous
import functools

import jax
import jax.numpy as jnp
import numpy as np
from jax import lax
from jax.experimental import pallas as pl
from jax.experimental.pallas import tpu as pltpu
from jax.experimental.pallas import tpu_sc as plsc

C = 12
CP = 16
D = H = W = 160
NPAD = 1 << 20

NC = 2
NS = 16
NW = NC * NS
PW = NPAD // NW
G = 128
NCH = PW // G

_SX = np.float32(0.5 * (W - 1))
_SY = np.float32(0.5 * (H - 1))
_SZ = np.float32(0.5 * (D - 1))


def _sc_body(xs, ys, zs, table, out, px_v, py_v, pz_v, idx_v, w_v, rows_v,
             out_v, sem):
    wid = lax.axis_index("s") * NC + lax.axis_index("c")
    wbase = wid * PW

    @pl.loop(0, NCH)
    def _chunk(g):
        base = wbase + g * G
        pltpu.sync_copy(xs.at[pl.ds(base, G)], px_v)
        pltpu.sync_copy(ys.at[pl.ds(base, G)], py_v)
        pltpu.sync_copy(zs.at[pl.ds(base, G)], pz_v)

        for j in range(G // 16):
            sl = pl.ds(j * 16, 16)
            fx = px_v[sl] * _SX + _SX
            fy = py_v[sl] * _SY + _SY
            fz = pz_v[sl] * _SZ + _SZ
            x0 = jnp.minimum(fx.astype(jnp.int32), W - 1)
            y0 = jnp.minimum(fy.astype(jnp.int32), H - 1)
            z0 = jnp.minimum(fz.astype(jnp.int32), D - 1)
            w_v[0, sl] = fx - x0.astype(jnp.float32)
            w_v[1, sl] = fy - y0.astype(jnp.float32)
            w_v[2, sl] = fz - z0.astype(jnp.float32)
            x1 = jnp.minimum(x0 + 1, W - 1)
            y1 = jnp.minimum(y0 + 1, H - 1)
            z1 = jnp.minimum(z0 + 1, D - 1)
            zb0 = z0 * (H * W)
            zb1 = z1 * (H * W)
            yb0 = y0 * W
            yb1 = y1 * W
            idx_v[0, sl] = zb0 + yb0 + x0
            idx_v[1, sl] = zb0 + yb0 + x1
            idx_v[2, sl] = zb0 + yb1 + x0
            idx_v[3, sl] = zb0 + yb1 + x1
            idx_v[4, sl] = zb1 + yb0 + x0
            idx_v[5, sl] = zb1 + yb0 + x1
            idx_v[6, sl] = zb1 + yb1 + x0
            idx_v[7, sl] = zb1 + yb1 + x1

        cps = [pltpu.make_async_copy(table.at[idx_v.at[t]], rows_v.at[t], sem)
               for t in range(8)]
        for cp in cps:
            cp.start()
        for cp in cps:
            cp.wait()

        @pl.loop(0, G // 16)
        def _grp(j):
            sl = pl.ds(j * 16, 16)
            wxv = w_v[0, sl]
            wyv = w_v[1, sl]
            wzv = w_v[2, sl]
            for k in range(16):
                p = j * 16 + k
                wx = wxv[k]
                wy = wyv[k]
                wz = wzv[k]
                c000 = rows_v[0, p]
                c001 = rows_v[1, p]
                c010 = rows_v[2, p]
                c011 = rows_v[3, p]
                c100 = rows_v[4, p]
                c101 = rows_v[5, p]
                c110 = rows_v[6, p]
                c111 = rows_v[7, p]
                a00 = c000 + wx * (c001 - c000)
                a01 = c010 + wx * (c011 - c010)
                a10 = c100 + wx * (c101 - c100)
                a11 = c110 + wx * (c111 - c110)
                b0 = a00 + wy * (a01 - a00)
                b1 = a10 + wy * (a11 - a10)
                out_v[p] = b0 + wz * (b1 - b0)

        pltpu.sync_copy(out_v, out.at[pl.ds(base, G)])


@jax.jit
def _run(xs, ys, zs, table):
    kern = pl.kernel(
        _sc_body,
        out_type=jax.ShapeDtypeStruct((NPAD, CP), jnp.float32),
        mesh=plsc.VectorSubcoreMesh(core_axis_name="c", subcore_axis_name="s"),
        scratch_types=[
            pltpu.VMEM((G,), jnp.float32),
            pltpu.VMEM((G,), jnp.float32),
            pltpu.VMEM((G,), jnp.float32),
            pltpu.VMEM((8, G), jnp.int32),
            pltpu.VMEM((3, G), jnp.float32),
            pltpu.VMEM((8, G, CP), jnp.float32),
            pltpu.VMEM((G, CP), jnp.float32),
            pltpu.SemaphoreType.DMA,
        ],
        compiler_params=pltpu.CompilerParams(use_tc_tiling_on_sc=False),
    )
    return kern(xs, ys, zs, table)


def kernel(ray_pts, k0):
    n = ray_pts.shape[0]
    table = jnp.pad(jnp.transpose(k0[0], (1, 2, 3, 0)),
                    ((0, 0), (0, 0), (0, 0), (0, CP - C)))
    table = table.reshape(D * H * W, CP)
    pts = jnp.pad(ray_pts, ((0, NPAD - n), (0, 0)))
    out = _run(pts[:, 0], pts[:, 1], pts[:, 2], table)
    return out[:n, :C]


# speedup vs baseline: 1.2266x; 1.1141x over previous
import functools

import jax
import jax.numpy as jnp
import numpy as np
from jax import lax
from jax.experimental import pallas as pl
from jax.experimental.pallas import tpu as pltpu
from jax.experimental.pallas import tpu_sc as plsc

C = 12
CP = 16
D = H = W = 160
NPAD = 1 << 20

NC = 2
NS = 16
NW = NC * NS
PW = NPAD // NW
G = 128
NCH = PW // G

_SX = np.float32(0.5 * (W - 1))
_SY = np.float32(0.5 * (H - 1))
_SZ = np.float32(0.5 * (D - 1))


def _sc_body(xs, ys, zs, table, out, pts_v, idx_v, w_v, rows_v, out_v,
             psem0, psem1, gsem0, gsem1, osem0, osem1):
    psem = (psem0, psem1)
    gsem = (gsem0, gsem1)
    osem = (osem0, osem1)
    wid = lax.axis_index("s") * NC + lax.axis_index("c")
    wbase = wid * PW

    def pts_copies(ch, b):
        base = wbase + ch * G
        return [
            pltpu.make_async_copy(xs.at[pl.ds(base, G)], pts_v.at[b, 0], psem[b]),
            pltpu.make_async_copy(ys.at[pl.ds(base, G)], pts_v.at[b, 1], psem[b]),
            pltpu.make_async_copy(zs.at[pl.ds(base, G)], pts_v.at[b, 2], psem[b]),
        ]

    def gather_copies(b):
        return [
            pltpu.make_async_copy(table.at[idx_v.at[b, t]], rows_v.at[b, t],
                                  gsem[b])
            for t in range(8)
        ]

    def out_copy(ch, b):
        base = (wbase + ch * G) * CP
        return pltpu.make_async_copy(out_v.at[b], out.at[pl.ds(base, G * CP)],
                                     osem[b])

    def compute_idx(b):
        for j in range(G // 16):
            sl = pl.ds(j * 16, 16)
            fx = pts_v[b, 0, sl] * _SX + _SX
            fy = pts_v[b, 1, sl] * _SY + _SY
            fz = pts_v[b, 2, sl] * _SZ + _SZ
            x0 = jnp.minimum(fx.astype(jnp.int32), W - 1)
            y0 = jnp.minimum(fy.astype(jnp.int32), H - 1)
            z0 = jnp.minimum(fz.astype(jnp.int32), D - 1)
            w_v[b, 0, sl] = fx - x0.astype(jnp.float32)
            w_v[b, 1, sl] = fy - y0.astype(jnp.float32)
            w_v[b, 2, sl] = fz - z0.astype(jnp.float32)
            x1 = jnp.minimum(x0 + 1, W - 1)
            y1 = jnp.minimum(y0 + 1, H - 1)
            z1 = jnp.minimum(z0 + 1, D - 1)
            zb0 = z0 * (H * W)
            zb1 = z1 * (H * W)
            yb0 = y0 * W
            yb1 = y1 * W
            idx_v[b, 0, sl] = zb0 + yb0 + x0
            idx_v[b, 1, sl] = zb0 + yb0 + x1
            idx_v[b, 2, sl] = zb0 + yb1 + x0
            idx_v[b, 3, sl] = zb0 + yb1 + x1
            idx_v[b, 4, sl] = zb1 + yb0 + x0
            idx_v[b, 5, sl] = zb1 + yb0 + x1
            idx_v[b, 6, sl] = zb1 + yb1 + x0
            idx_v[b, 7, sl] = zb1 + yb1 + x1

    def interp(b):
        @pl.loop(0, G // 16)
        def _grp(j):
            sl = pl.ds(j * 16, 16)
            wxv = w_v[b, 0, sl]
            wyv = w_v[b, 1, sl]
            wzv = w_v[b, 2, sl]
            for k in range(16):
                p = j * 16 + k
                wx = wxv[k]
                wy = wyv[k]
                wz = wzv[k]
                c000 = rows_v[b, 0, p]
                c001 = rows_v[b, 1, p]
                c010 = rows_v[b, 2, p]
                c011 = rows_v[b, 3, p]
                c100 = rows_v[b, 4, p]
                c101 = rows_v[b, 5, p]
                c110 = rows_v[b, 6, p]
                c111 = rows_v[b, 7, p]
                a00 = c000 + wx * (c001 - c000)
                a01 = c010 + wx * (c011 - c010)
                a10 = c100 + wx * (c101 - c100)
                a11 = c110 + wx * (c111 - c110)
                b0 = a00 + wy * (a01 - a00)
                b1 = a10 + wy * (a11 - a10)
                out_v[b, pl.ds(p * CP, CP)] = b0 + wz * (b1 - b0)

    for cp in pts_copies(0, 0):
        cp.start()
    for cp in pts_copies(1, 1):
        cp.start()
    for cp in pts_copies(0, 0):
        cp.wait()
    compute_idx(0)
    for cp in gather_copies(0):
        cp.start()

    @pl.loop(0, NCH, step=2)
    def _pair(g):
        for b in (0, 1):
            ch = g + b
            nb = 1 - b
            @pl.when(ch + 1 < NCH)
            def _stage():
                for cp in pts_copies(ch + 1, nb):
                    cp.wait()
                compute_idx(nb)
                for cp in gather_copies(nb):
                    cp.start()

            @pl.when(ch + 2 < NCH)
            def _prefetch():
                for cp in pts_copies(ch + 2, b):
                    cp.start()

            for cp in gather_copies(b):
                cp.wait()

            @pl.when(ch >= 2)
            def _reclaim():
                out_copy(ch - 2, b).wait()

            interp(b)
            out_copy(ch, b).start()

    out_copy(NCH - 2, 0).wait()
    out_copy(NCH - 1, 1).wait()


@jax.jit
def _run(xs, ys, zs, table):
    kern = pl.kernel(
        _sc_body,
        out_type=jax.ShapeDtypeStruct((NPAD * CP,), jnp.float32),
        mesh=plsc.VectorSubcoreMesh(core_axis_name="c", subcore_axis_name="s"),
        scratch_types=[
            pltpu.VMEM((2, 3, G), jnp.float32),
            pltpu.VMEM((2, 8, G), jnp.int32),
            pltpu.VMEM((2, 3, G), jnp.float32),
            pltpu.VMEM((2, 8, G, CP), jnp.float32),
            pltpu.VMEM((2, G * CP), jnp.float32),
            pltpu.SemaphoreType.DMA,
            pltpu.SemaphoreType.DMA,
            pltpu.SemaphoreType.DMA,
            pltpu.SemaphoreType.DMA,
            pltpu.SemaphoreType.DMA,
            pltpu.SemaphoreType.DMA,
        ],
        compiler_params=pltpu.CompilerParams(use_tc_tiling_on_sc=False),
    )
    return kern(xs, ys, zs, table)


def kernel(ray_pts, k0):
    n = ray_pts.shape[0]
    table = jnp.pad(jnp.transpose(k0[0], (1, 2, 3, 0)),
                    ((0, 0), (0, 0), (0, 0), (0, CP - C)))
    table = table.reshape(D * H * W, CP)
    pts = jnp.pad(ray_pts, ((0, NPAD - n), (0, 0)))
    out = _run(pts[:, 0], pts[:, 1], pts[:, 2], table)
    return out.reshape(NPAD, CP)[:n, :C]
